# baseline (speedup 1.0000x reference)
import functools

import jax
import jax.numpy as jnp
from jax import lax
from jax.experimental import pallas as pl
from jax.experimental.pallas import tpu as pltpu
from jax.experimental.pallas import tpu_sc as plsc

ENC_DIM = 1024
PAD = 1
BATCH = 4
SEQ = 4096
N = BATCH * SEQ
NUM_CORES = 2
NUM_SUBCORES = 16
NUM_WORKERS = NUM_CORES * NUM_SUBCORES
CHUNK = N // NUM_WORKERS
TILES_PER_ROW = SEQ // CHUNK
GROWS = 32
NGC = CHUNK // GROWS
NBUF = 3
LANES = 16


def _body(x_hbm, w_hbm, out_hbm, xrow, idxv, buf0, buf1, buf2, gsem, ssem):
    c = lax.axis_index("c")
    s = lax.axis_index("s")
    g = c * NUM_SUBCORES + s
    base = g * CHUNK
    r = g // TILES_PER_ROW
    j = g % TILES_PER_ROW

    pltpu.sync_copy(x_hbm.at[pl.ds(r * SEQ, SEQ)], xrow)

    def cnt_body(i, acc):
        v = xrow[pl.ds(i * LANES, LANES)]
        return acc + jnp.where(v != PAD, jnp.int32(1), jnp.int32(0))

    npre = j * (CHUNK // LANES)
    acc = lax.fori_loop(0, npre, cnt_body, jnp.zeros((LANES,), jnp.int32))
    prefix = jnp.sum(acc)

    bufs = [buf0, buf1, buf2]
    coff = j * CHUNK

    def start_gather(k):
        return pltpu.async_copy(
            w_hbm.at[idxv.at[pl.ds(k * GROWS, GROWS)]], bufs[k % NBUF], gsem)

    def start_scatter(k):
        return pltpu.async_copy(
            bufs[k % NBUF], out_hbm.at[pl.ds(base + k * GROWS, GROWS)], ssem)

    gathers = [None] * NGC
    scatters = [None] * NGC
    run = prefix
    for k in range(NGC):
        for i2 in range(GROWS // LANES):
            v = xrow[pl.ds(coff + k * GROWS + i2 * LANES, LANES)]
            m = jnp.where(v != PAD, jnp.int32(1), jnp.int32(0))
            cs = jnp.cumsum(m)
            idxv[pl.ds(k * GROWS + i2 * LANES, LANES)] = (cs + run) * m + PAD
            run = run + jnp.sum(m)
        if k >= NBUF:
            scatters[k - NBUF].wait()
        gathers[k] = start_gather(k)
        if k >= NBUF - 1:
            kk = k - (NBUF - 1)
            gathers[kk].wait()
            scatters[kk] = start_scatter(kk)
    for kk in range(NGC - (NBUF - 1), NGC):
        gathers[kk].wait()
        scatters[kk] = start_scatter(kk)
    for kk in range(NGC - NBUF, NGC):
        scatters[kk].wait()


@functools.partial(jax.jit)
def kernel(x, weights):
    x_flat = x.reshape(-1).astype(jnp.int32)
    weights = weights.astype(jnp.float32)
    mesh = plsc.VectorSubcoreMesh(core_axis_name="c", subcore_axis_name="s")
    run = pl.kernel(
        _body,
        mesh=mesh,
        compiler_params=pltpu.CompilerParams(needs_layout_passes=False),
        out_type=jax.ShapeDtypeStruct((N, ENC_DIM), jnp.float32),
        scratch_types=[
            pltpu.VMEM((SEQ,), jnp.int32),
            pltpu.VMEM((CHUNK,), jnp.int32),
            pltpu.VMEM((GROWS, ENC_DIM), jnp.float32),
            pltpu.VMEM((GROWS, ENC_DIM), jnp.float32),
            pltpu.VMEM((GROWS, ENC_DIM), jnp.float32),
            pltpu.SemaphoreType.DMA,
            pltpu.SemaphoreType.DMA,
        ],
    )
    out = run(x_flat, weights)
    return lax.stop_gradient(out.reshape(BATCH, SEQ, ENC_DIM))


# speedup vs baseline: 1.4611x; 1.4611x over previous
import functools

import jax
import jax.numpy as jnp
from jax import lax
from jax.experimental import pallas as pl
from jax.experimental.pallas import tpu as pltpu
from jax.experimental.pallas import tpu_sc as plsc

ENC_DIM = 1024
PAD = 1
BATCH = 4
SEQ = 4096
N = BATCH * SEQ
NUM_CORES = 2
NUM_SUBCORES = 16
NUM_WORKERS = NUM_CORES * NUM_SUBCORES
CHUNK = N // NUM_WORKERS
TILES_PER_ROW = SEQ // CHUNK
GROWS = 32
NGC = CHUNK // GROWS
LANES = 16
HBLK = 512
HJ = SEQ // HBLK


def _h_body(w_hbm, o_ref, vba, vbb, sema, semb):
    j = pl.program_id(0)

    def fire(jj, vb, sem):
        pltpu.async_copy(w_hbm.at[pl.ds(jj * HBLK, HBLK + 8)], vb, sem)

    def wait(jj, vb, sem):
        pltpu.make_async_copy(
            w_hbm.at[pl.ds(jj * HBLK, HBLK + 8)], vb, sem).wait()

    @pl.when(j == 0)
    def _():
        fire(j, vba, sema)

    @pl.when(jnp.logical_and(j + 1 < HJ, (j + 1) % 2 == 0))
    def _():
        fire(j + 1, vba, sema)

    @pl.when(jnp.logical_and(j + 1 < HJ, (j + 1) % 2 == 1))
    def _():
        fire(j + 1, vbb, semb)

    def shifted_copy(vb):
        for t in range(0, HBLK, GROWS):
            v = vb[pl.ds(2 + t, GROWS), :]
            for b in range(BATCH):
                o_ref[b, pl.ds(t, GROWS), :] = v

    @pl.when(j % 2 == 0)
    def _():
        wait(j, vba, sema)
        shifted_copy(vba)

    @pl.when(j % 2 == 1)
    def _():
        wait(j, vbb, semb)
        shifted_copy(vbb)


def _run_h(weights):
    return pl.pallas_call(
        _h_body,
        grid=(HJ,),
        in_specs=[pl.BlockSpec(memory_space=pl.ANY)],
        out_specs=pl.BlockSpec((BATCH, HBLK, ENC_DIM), lambda j: (0, j, 0)),
        out_shape=jax.ShapeDtypeStruct((BATCH, SEQ, ENC_DIM), jnp.float32),
        scratch_shapes=[
            pltpu.VMEM((HBLK + 8, ENC_DIM), jnp.float32),
            pltpu.VMEM((HBLK + 8, ENC_DIM), jnp.float32),
            pltpu.SemaphoreType.DMA,
            pltpu.SemaphoreType.DMA,
        ],
    )(weights)


def _p_body(x_hbm, idx_hbm, xrow, idxv):
    c = lax.axis_index("c")
    s = lax.axis_index("s")
    g = c * NUM_SUBCORES + s
    base = g * CHUNK
    r = g // TILES_PER_ROW
    j = g % TILES_PER_ROW

    pltpu.sync_copy(x_hbm.at[pl.ds(r * SEQ, SEQ)], xrow)

    def cnt_body(i, acc):
        v = xrow[pl.ds(i * LANES, LANES)]
        return acc + jnp.where(v != PAD, jnp.int32(1), jnp.int32(0))

    npre = j * (CHUNK // LANES)
    acc = lax.fori_loop(0, npre, cnt_body, jnp.zeros((LANES,), jnp.int32))
    run = jnp.sum(acc)

    coff = j * CHUNK
    for i in range(CHUNK // LANES):
        v = xrow[pl.ds(coff + i * LANES, LANES)]
        m = jnp.where(v != PAD, jnp.int32(1), jnp.int32(0))
        cs = jnp.cumsum(m)
        idxv[pl.ds(i * LANES, LANES)] = (cs + run) * m + PAD
        run = run + jnp.sum(m)
    pltpu.sync_copy(idxv, idx_hbm.at[pl.ds(base, CHUNK)])


def _run_p(x_flat):
    mesh = plsc.VectorSubcoreMesh(core_axis_name="c", subcore_axis_name="s")
    return pl.kernel(
        _p_body,
        mesh=mesh,
        compiler_params=pltpu.CompilerParams(needs_layout_passes=False),
        out_type=jax.ShapeDtypeStruct((N,), jnp.int32),
        scratch_types=[
            pltpu.VMEM((SEQ,), jnp.int32),
            pltpu.VMEM((CHUNK,), jnp.int32),
        ],
    )(x_flat)


def _c3_body(idx_hbm, w_hbm, out_hbm, idxv, buf, gsem, ssem):
    c = lax.axis_index("c")
    s = lax.axis_index("s")
    g = c * NUM_SUBCORES + s
    base = g * CHUNK
    r = g // TILES_PER_ROW
    t0 = (g % TILES_PER_ROW) * CHUNK

    pltpu.sync_copy(idx_hbm.at[pl.ds(base, CHUNK)], idxv)
    lane = lax.iota(jnp.int32, LANES)

    for k in range(NGC):
        nbad = jnp.zeros((LANES,), jnp.int32)
        for i2 in range(GROWS // LANES):
            v = idxv[pl.ds(k * GROWS + i2 * LANES, LANES)]
            expect = lane + (t0 + k * GROWS + i2 * LANES + 2)
            nbad = nbad + jnp.where(v != expect, jnp.int32(1), jnp.int32(0))
        bad = jnp.sum(nbad) > 0

        @pl.when(bad)
        def _(k=k):
            pltpu.async_copy(
                w_hbm.at[idxv.at[pl.ds(k * GROWS, GROWS)]], buf, gsem).wait()
            pltpu.async_copy(
                buf,
                out_hbm.at[r].at[pl.ds(t0 + k * GROWS, GROWS)], ssem).wait()


def _run_c3(idx, weights, out_ref):
    mesh = plsc.VectorSubcoreMesh(core_axis_name="c", subcore_axis_name="s")
    return pl.kernel(
        _c3_body,
        mesh=mesh,
        compiler_params=pltpu.CompilerParams(needs_layout_passes=False),
        out_type=(),
        scratch_types=[
            pltpu.VMEM((CHUNK,), jnp.int32),
            pltpu.VMEM((GROWS, ENC_DIM), jnp.float32),
            pltpu.SemaphoreType.DMA,
            pltpu.SemaphoreType.DMA,
        ],
    )(idx, weights, out_ref)


@functools.partial(jax.jit)
def kernel(x, weights):
    x_flat = x.reshape(-1).astype(jnp.int32)
    weights = weights.astype(jnp.float32)
    hyp = _run_h(weights)
    idx = _run_p(x_flat)
    out_ref = jax.new_ref(hyp)
    _run_c3(idx, weights, out_ref)
    return lax.stop_gradient(jax.freeze(out_ref))


# speedup vs baseline: 1.4954x; 1.0235x over previous
import functools

import jax
import jax.numpy as jnp
from jax import lax
from jax.experimental import pallas as pl
from jax.experimental.pallas import tpu as pltpu
from jax.experimental.pallas import tpu_sc as plsc

ENC_DIM = 1024
PAD = 1
BATCH = 4
SEQ = 4096
N = BATCH * SEQ
NUM_CORES = 2
NUM_SUBCORES = 16
NUM_WORKERS = NUM_CORES * NUM_SUBCORES
CHUNK = N // NUM_WORKERS
TILES_PER_ROW = SEQ // CHUNK
GROWS = 32
NGC = CHUNK // GROWS
LANES = 16
HBLK = 1024
HJ = SEQ // HBLK


def _h_body(w_hbm, o_ref, vba, vbb, sema, semb):
    j = pl.program_id(0)

    def fire(jj, vb, sem):
        pltpu.async_copy(w_hbm.at[pl.ds(jj * HBLK, HBLK + 8)], vb, sem)

    def wait(jj, vb, sem):
        pltpu.make_async_copy(
            w_hbm.at[pl.ds(jj * HBLK, HBLK + 8)], vb, sem).wait()

    @pl.when(j == 0)
    def _():
        fire(j, vba, sema)

    @pl.when(jnp.logical_and(j + 1 < HJ, (j + 1) % 2 == 0))
    def _():
        fire(j + 1, vba, sema)

    @pl.when(jnp.logical_and(j + 1 < HJ, (j + 1) % 2 == 1))
    def _():
        fire(j + 1, vbb, semb)

    def shifted_copy(vb):
        for t in range(0, HBLK, GROWS):
            v = vb[pl.ds(2 + t, GROWS), :]
            for b in range(BATCH):
                o_ref[b, pl.ds(t, GROWS), :] = v

    @pl.when(j % 2 == 0)
    def _():
        wait(j, vba, sema)
        shifted_copy(vba)

    @pl.when(j % 2 == 1)
    def _():
        wait(j, vbb, semb)
        shifted_copy(vbb)


def _run_h(weights):
    return pl.pallas_call(
        _h_body,
        grid=(HJ,),
        in_specs=[pl.BlockSpec(memory_space=pl.ANY)],
        out_specs=pl.BlockSpec((BATCH, HBLK, ENC_DIM), lambda j: (0, j, 0)),
        out_shape=jax.ShapeDtypeStruct((BATCH, SEQ, ENC_DIM), jnp.float32),
        scratch_shapes=[
            pltpu.VMEM((HBLK + 8, ENC_DIM), jnp.float32),
            pltpu.VMEM((HBLK + 8, ENC_DIM), jnp.float32),
            pltpu.SemaphoreType.DMA,
            pltpu.SemaphoreType.DMA,
        ],
    )(weights)


def _p_body(x_hbm, idx_hbm, xrow, idxv):
    c = lax.axis_index("c")
    s = lax.axis_index("s")
    g = c * NUM_SUBCORES + s
    base = g * CHUNK
    r = g // TILES_PER_ROW
    j = g % TILES_PER_ROW

    pltpu.sync_copy(x_hbm.at[pl.ds(r * SEQ, SEQ)], xrow)

    def cnt_body(i, acc):
        v = xrow[pl.ds(i * LANES, LANES)]
        return acc + jnp.where(v != PAD, jnp.int32(1), jnp.int32(0))

    npre = j * (CHUNK // LANES)
    acc = lax.fori_loop(0, npre, cnt_body, jnp.zeros((LANES,), jnp.int32))
    run = jnp.sum(acc)

    coff = j * CHUNK
    for i in range(CHUNK // LANES):
        v = xrow[pl.ds(coff + i * LANES, LANES)]
        m = jnp.where(v != PAD, jnp.int32(1), jnp.int32(0))
        cs = jnp.cumsum(m)
        idxv[pl.ds(i * LANES, LANES)] = (cs + run) * m + PAD
        run = run + jnp.sum(m)
    pltpu.sync_copy(idxv, idx_hbm.at[pl.ds(base, CHUNK)])


def _run_p(x_flat):
    mesh = plsc.VectorSubcoreMesh(core_axis_name="c", subcore_axis_name="s")
    return pl.kernel(
        _p_body,
        mesh=mesh,
        compiler_params=pltpu.CompilerParams(needs_layout_passes=False),
        out_type=jax.ShapeDtypeStruct((N,), jnp.int32),
        scratch_types=[
            pltpu.VMEM((SEQ,), jnp.int32),
            pltpu.VMEM((CHUNK,), jnp.int32),
        ],
    )(x_flat)


def _c3_body(idx_hbm, w_hbm, out_hbm, idxv, buf, gsem, ssem):
    c = lax.axis_index("c")
    s = lax.axis_index("s")
    g = c * NUM_SUBCORES + s
    base = g * CHUNK
    r = g // TILES_PER_ROW
    t0 = (g % TILES_PER_ROW) * CHUNK

    pltpu.sync_copy(idx_hbm.at[pl.ds(base, CHUNK)], idxv)
    lane = lax.iota(jnp.int32, LANES)

    for k in range(NGC):
        nbad = jnp.zeros((LANES,), jnp.int32)
        for i2 in range(GROWS // LANES):
            v = idxv[pl.ds(k * GROWS + i2 * LANES, LANES)]
            expect = lane + (t0 + k * GROWS + i2 * LANES + 2)
            nbad = nbad + jnp.where(v != expect, jnp.int32(1), jnp.int32(0))
        bad = jnp.sum(nbad) > 0

        @pl.when(bad)
        def _(k=k):
            pltpu.async_copy(
                w_hbm.at[idxv.at[pl.ds(k * GROWS, GROWS)]], buf, gsem).wait()
            pltpu.async_copy(
                buf,
                out_hbm.at[r].at[pl.ds(t0 + k * GROWS, GROWS)], ssem).wait()


def _run_c3(idx, weights, out_ref):
    mesh = plsc.VectorSubcoreMesh(core_axis_name="c", subcore_axis_name="s")
    return pl.kernel(
        _c3_body,
        mesh=mesh,
        compiler_params=pltpu.CompilerParams(needs_layout_passes=False),
        out_type=(),
        scratch_types=[
            pltpu.VMEM((CHUNK,), jnp.int32),
            pltpu.VMEM((GROWS, ENC_DIM), jnp.float32),
            pltpu.SemaphoreType.DMA,
            pltpu.SemaphoreType.DMA,
        ],
    )(idx, weights, out_ref)


@functools.partial(jax.jit)
def kernel(x, weights):
    x_flat = x.reshape(-1).astype(jnp.int32)
    weights = weights.astype(jnp.float32)
    hyp = _run_h(weights)
    idx = _run_p(x_flat)
    out_ref = jax.new_ref(hyp)
    _run_c3(idx, weights, out_ref)
    return lax.stop_gradient(jax.freeze(out_ref))


# speedup vs baseline: 1.4954x; 1.0000x over previous
import functools

import jax
import jax.numpy as jnp
from jax import lax
from jax.experimental import pallas as pl
from jax.experimental.pallas import tpu as pltpu
from jax.experimental.pallas import tpu_sc as plsc

ENC_DIM = 1024
PAD = 1
BATCH = 4
SEQ = 4096
N = BATCH * SEQ
NUM_CORES = 2
NUM_SUBCORES = 16
NUM_WORKERS = NUM_CORES * NUM_SUBCORES
CHUNK = N // NUM_WORKERS
TILES_PER_ROW = SEQ // CHUNK
GROWS = 32
NGC = CHUNK // GROWS
LANES = 16
HBLK = 1024
HJ = SEQ // HBLK


def _h_body(w_hbm, o_ref, vba, vbb, sema, semb):
    j = pl.program_id(0)

    def fire(jj, vb, sem):
        pltpu.async_copy(w_hbm.at[pl.ds(jj * HBLK, HBLK + 8)], vb, sem)

    def wait(jj, vb, sem):
        pltpu.make_async_copy(
            w_hbm.at[pl.ds(jj * HBLK, HBLK + 8)], vb, sem).wait()

    @pl.when(j == 0)
    def _():
        fire(j, vba, sema)

    @pl.when(jnp.logical_and(j + 1 < HJ, (j + 1) % 2 == 0))
    def _():
        fire(j + 1, vba, sema)

    @pl.when(jnp.logical_and(j + 1 < HJ, (j + 1) % 2 == 1))
    def _():
        fire(j + 1, vbb, semb)

    def shifted_copy(vb):
        for t in range(0, HBLK, GROWS):
            v = vb[pl.ds(2 + t, GROWS), :]
            for b in range(BATCH):
                o_ref[b, pl.ds(t, GROWS), :] = v

    @pl.when(j % 2 == 0)
    def _():
        wait(j, vba, sema)
        shifted_copy(vba)

    @pl.when(j % 2 == 1)
    def _():
        wait(j, vbb, semb)
        shifted_copy(vbb)


def _run_h(weights):
    return pl.pallas_call(
        _h_body,
        grid=(HJ,),
        in_specs=[pl.BlockSpec(memory_space=pl.ANY)],
        out_specs=pl.BlockSpec((BATCH, HBLK, ENC_DIM), lambda j: (0, j, 0)),
        out_shape=jax.ShapeDtypeStruct((BATCH, SEQ, ENC_DIM), jnp.float32),
        scratch_shapes=[
            pltpu.VMEM((HBLK + 8, ENC_DIM), jnp.float32),
            pltpu.VMEM((HBLK + 8, ENC_DIM), jnp.float32),
            pltpu.SemaphoreType.DMA,
            pltpu.SemaphoreType.DMA,
        ],
    )(weights)


def _p_body(x_hbm, idx_hbm, xrow, idxv):
    c = lax.axis_index("c")
    s = lax.axis_index("s")
    g = c * NUM_SUBCORES + s
    base = g * CHUNK
    r = g // TILES_PER_ROW
    j = g % TILES_PER_ROW

    pltpu.sync_copy(x_hbm.at[pl.ds(r * SEQ, SEQ)], xrow)

    def cnt_body(i, acc):
        v = xrow[pl.ds(i * LANES, LANES)]
        return acc + jnp.where(v != PAD, jnp.int32(1), jnp.int32(0))

    npre = j * (CHUNK // LANES)
    acc = lax.fori_loop(0, npre, cnt_body, jnp.zeros((LANES,), jnp.int32))
    run = jnp.sum(acc)

    coff = j * CHUNK
    for i in range(CHUNK // LANES):
        v = xrow[pl.ds(coff + i * LANES, LANES)]
        m = jnp.where(v != PAD, jnp.int32(1), jnp.int32(0))
        cs = jnp.cumsum(m)
        idxv[pl.ds(i * LANES, LANES)] = (cs + run) * m + PAD
        run = run + jnp.sum(m)
    pltpu.sync_copy(idxv, idx_hbm.at[pl.ds(base, CHUNK)])


def _run_p(x_flat):
    mesh = plsc.VectorSubcoreMesh(core_axis_name="c", subcore_axis_name="s")
    return pl.kernel(
        _p_body,
        mesh=mesh,
        compiler_params=pltpu.CompilerParams(
            needs_layout_passes=False, skip_device_barrier=True),
        out_type=jax.ShapeDtypeStruct((N,), jnp.int32),
        scratch_types=[
            pltpu.VMEM((SEQ,), jnp.int32),
            pltpu.VMEM((CHUNK,), jnp.int32),
        ],
    )(x_flat)


def _c3_body(idx_hbm, w_hbm, out_hbm, idxv, buf, gsem, ssem):
    c = lax.axis_index("c")
    s = lax.axis_index("s")
    g = c * NUM_SUBCORES + s
    base = g * CHUNK
    r = g // TILES_PER_ROW
    t0 = (g % TILES_PER_ROW) * CHUNK

    pltpu.sync_copy(idx_hbm.at[pl.ds(base, CHUNK)], idxv)
    lane = lax.iota(jnp.int32, LANES)

    for k in range(NGC):
        nbad = jnp.zeros((LANES,), jnp.int32)
        for i2 in range(GROWS // LANES):
            v = idxv[pl.ds(k * GROWS + i2 * LANES, LANES)]
            expect = lane + (t0 + k * GROWS + i2 * LANES + 2)
            nbad = nbad + jnp.where(v != expect, jnp.int32(1), jnp.int32(0))
        bad = jnp.sum(nbad) > 0

        @pl.when(bad)
        def _(k=k):
            pltpu.async_copy(
                w_hbm.at[idxv.at[pl.ds(k * GROWS, GROWS)]], buf, gsem).wait()
            pltpu.async_copy(
                buf,
                out_hbm.at[r].at[pl.ds(t0 + k * GROWS, GROWS)], ssem).wait()


def _run_c3(idx, weights, out_ref):
    mesh = plsc.VectorSubcoreMesh(core_axis_name="c", subcore_axis_name="s")
    return pl.kernel(
        _c3_body,
        mesh=mesh,
        compiler_params=pltpu.CompilerParams(
            needs_layout_passes=False, skip_device_barrier=True),
        out_type=(),
        scratch_types=[
            pltpu.VMEM((CHUNK,), jnp.int32),
            pltpu.VMEM((GROWS, ENC_DIM), jnp.float32),
            pltpu.SemaphoreType.DMA,
            pltpu.SemaphoreType.DMA,
        ],
    )(idx, weights, out_ref)


@functools.partial(jax.jit)
def kernel(x, weights):
    x_flat = x.reshape(-1).astype(jnp.int32)
    weights = weights.astype(jnp.float32)
    hyp = _run_h(weights)
    idx = _run_p(x_flat)
    out_ref = jax.new_ref(hyp)
    _run_c3(idx, weights, out_ref)
    return lax.stop_gradient(jax.freeze(out_ref))
